# baseline (speedup 1.0000x reference)
import functools

import jax
import jax.numpy as jnp
from jax import lax
from jax.experimental import pallas as pl
from jax.experimental.pallas import tpu as pltpu
from jax.experimental.pallas import tpu_sc as plsc


def _sc_ctx_sum(xflat, ctx, batch, emb_table, base, n, n_workers=32,
                num_cores=2):
    _, d = emb_table.shape
    nb = n // n_workers

    mesh = plsc.VectorSubcoreMesh(core_axis_name="c", subcore_axis_name="s")

    @functools.partial(
        pl.kernel,
        out_type=jax.ShapeDtypeStruct((n, d), jnp.float32),
        mesh=mesh,
        scratch_types=[
            pltpu.VMEM((ctx, nb), jnp.int32),
            pltpu.VMEM((nb, d), jnp.float32),
            pltpu.SemaphoreType.DMA,
        ],
    )
    def sc_sum(xflat_hbm, table_hbm, out_hbm, idx_v, acc_v, sem):
        wid = lax.axis_index("s") * num_cores + lax.axis_index("c")
        off = wid * nb
        idx_cps = [
            pltpu.async_copy(
                xflat_hbm.at[pl.ds(c * batch + base + off, nb)],
                idx_v.at[c], sem)
            for c in range(ctx)
        ]
        for cp in idx_cps:
            cp.wait()
        pltpu.async_copy(table_hbm.at[idx_v.at[0]], acc_v, sem).wait()
        adds = [
            pltpu.async_copy(table_hbm.at[idx_v.at[c]], acc_v, sem, add=True)
            for c in range(1, ctx)
        ]
        for cp in adds:
            cp.wait()
        pltpu.sync_copy(acc_v, out_hbm.at[pl.ds(off, nb)])

    return sc_sum(xflat, emb_table)


def _tc_project_t(sums_t_bf16, W, brow, ctx, vocab, batch, col, out_alias=None,
                  vt=1024):
    d, n = sums_t_bf16.shape
    inv_ctx = 1.0 / ctx

    def body(*refs):
        s_ref, w_ref, b_ref, o_ref = refs[-4:]
        w = (w_ref[...] * inv_ctx).astype(jnp.bfloat16)
        o_ref[...] = lax.dot_general(
            w, s_ref[...], (((1,), (0,)), ((), ())),
            preferred_element_type=jnp.float32,
        ) + b_ref[...].T

    in_specs = [
        pl.BlockSpec((d, n), lambda j: (0, 0)),
        pl.BlockSpec((vt, d), lambda j: (j, 0)),
        pl.BlockSpec((1, vt), lambda j: (0, j)),
    ]
    operands = [sums_t_bf16, W, brow]
    aliases = {}
    if out_alias is not None:
        in_specs = [pl.BlockSpec(memory_space=pltpu.HBM)] + in_specs
        operands = [out_alias] + operands
        aliases = {0: 0}

    return pl.pallas_call(
        body,
        grid=(pl.cdiv(vocab, vt),),
        in_specs=in_specs,
        out_specs=pl.BlockSpec((vt, n), lambda j: (j, col)),
        out_shape=jax.ShapeDtypeStruct((vocab, batch), jnp.float32),
        input_output_aliases=aliases,
    )(*operands)


def kernel(x, emb_table, W, b):
    batch, ctx = x.shape
    vocab = W.shape[0]
    half = batch // 2
    xflat = x.T.reshape(-1)
    brow = b.reshape(1, -1)

    sums_a = _sc_ctx_sum(xflat, ctx, batch, emb_table, 0, half)
    sums_b = _sc_ctx_sum(xflat, ctx, batch, emb_table, half, half)
    out = _tc_project_t(sums_a.T.astype(jnp.bfloat16), W, brow, ctx,
                        vocab, batch, col=0)
    out = _tc_project_t(sums_b.T.astype(jnp.bfloat16), W, brow, ctx,
                        vocab, batch, col=1, out_alias=out)
    return out.T
